# baseline (speedup 1.0000x reference)
import functools

import jax
import jax.numpy as jnp
from jax import lax
from jax.experimental import pallas as pl
from jax.experimental.pallas import tpu as pltpu
from jax.experimental.pallas import tpu_sc as plsc

_NC = 2
_NS = 16
_NW = _NC * _NS
_CH = 128


@functools.lru_cache(maxsize=None)
def _make_edge_pass(with_count, n_chunks, n_acc, n, d):
    mesh = plsc.VectorSubcoreMesh(core_axis_name="c", subcore_axis_name="s")
    rps = n_acc // _NS

    out_type = [jax.ShapeDtypeStruct((_NC, n_acc, d), jnp.float32)]
    scratch = [
        pltpu.VMEM((n_chunks, _CH), jnp.int32),
        pltpu.VMEM((n_chunks, _CH), jnp.int32),
        pltpu.VMEM((_CH, d), jnp.float32),
        pltpu.VMEM((_CH, d), jnp.float32),
        pltpu.VMEM_SHARED((n_acc, d), jnp.float32),
        pltpu.SemaphoreType.DMA,
        pltpu.SemaphoreType.DMA,
        pltpu.SemaphoreType.DMA,
        pltpu.SemaphoreType.DMA,
    ]
    if with_count:
        out_type.append(jax.ShapeDtypeStruct((_NC, n_acc, 16), jnp.float32))
        scratch += [
            pltpu.VMEM((_CH, 16), jnp.float32),
            pltpu.VMEM_SHARED((n_acc, 16), jnp.float32),
        ]

    def striped(s, fn):
        fn(pl.ds(s * rps, rps))

    def body(y_ref, src_ref, dst_ref, z64_ref, *rest):
        if with_count:
            (z16_ref, part_ref, cnt_ref, src_v, dst_v, b0, b1, acc_sh,
             g0, g1, s0, s1, ones_v, cnt_sh) = rest
        else:
            (part_ref, src_v, dst_v, b0, b1, acc_sh,
             g0, g1, s0, s1) = rest
        c = lax.axis_index("c")
        s = lax.axis_index("s")
        wid = s * _NC + c

        striped(s, lambda sl: pltpu.sync_copy(z64_ref.at[sl], acc_sh.at[sl]))
        if with_count:
            striped(s, lambda sl: pltpu.sync_copy(z16_ref.at[sl],
                                                  cnt_sh.at[sl]))
            vec = jnp.where(lax.iota(jnp.int32, 16) == 0,
                            jnp.float32(1.0), jnp.float32(0.0))

            def init_ones(i, _):
                ones_v[i, :] = vec
                return 0
            lax.fori_loop(0, _CH, init_ones, 0)

        pltpu.sync_copy(src_ref.at[wid], src_v)
        pltpu.sync_copy(dst_ref.at[wid], dst_v)
        plsc.subcore_barrier()

        pltpu.async_copy(y_ref.at[src_v.at[0]], b0, g0)
        pltpu.async_copy(y_ref.at[src_v.at[1]], b1, g1)

        def pair(j, _):
            a = 2 * j
            for buf, gsem, ssem, k in ((b0, g0, s0, a), (b1, g1, s1, a + 1)):
                pltpu.make_async_copy(y_ref.at[src_v.at[k]], buf, gsem).wait()
                pltpu.async_copy(buf, acc_sh.at[dst_v.at[k]], ssem, add=True)
                if with_count:
                    pltpu.async_copy(ones_v, cnt_sh.at[dst_v.at[k]], ssem,
                                     add=True)
            for buf, gsem, ssem, k in ((b0, g0, s0, a), (b1, g1, s1, a + 1)):
                pltpu.make_async_copy(buf, acc_sh.at[dst_v.at[k]], ssem).wait()
                if with_count:
                    pltpu.make_async_copy(ones_v, cnt_sh.at[dst_v.at[k]],
                                          ssem).wait()

                @pl.when(k + 2 < n_chunks)
                def _():
                    pltpu.async_copy(y_ref.at[src_v.at[k + 2]], buf, gsem)
            return 0

        lax.fori_loop(0, n_chunks // 2, pair, 0)
        plsc.subcore_barrier()

        striped(s, lambda sl: pltpu.sync_copy(acc_sh.at[sl],
                                              part_ref.at[c, sl]))
        if with_count:
            striped(s, lambda sl: pltpu.sync_copy(cnt_sh.at[sl],
                                                  cnt_ref.at[c, sl]))

    return pl.kernel(body, out_type=tuple(out_type), mesh=mesh,
                     scratch_types=tuple(scratch),
                     compiler_params=pltpu.CompilerParams(
                         use_tc_tiling_on_sc=False))


def _mm2_body(x_ref, wl_ref, wr_ref, bl_ref, y_ref, z_ref):
    x = x_ref[...]
    y_ref[...] = jnp.dot(x, wl_ref[...], preferred_element_type=jnp.float32,
                         precision=lax.Precision.HIGHEST)
    z_ref[...] = (jnp.dot(x, wr_ref[...], preferred_element_type=jnp.float32)
                  + bl_ref[...][None, :])


_TC_PARAMS = pltpu.CompilerParams(vmem_limit_bytes=100 * 1024 * 1024)
_BR = 2000


def _row_spec(shape_tail):
    return pl.BlockSpec((_BR,) + shape_tail, lambda i: (i,) + (0,) * len(shape_tail))


def _part_spec(shape_tail):
    return pl.BlockSpec((2, _BR) + shape_tail,
                        lambda i: (0, i) + (0,) * len(shape_tail))


def _full_spec(ndim):
    return pl.BlockSpec(None, lambda i: (0,) * ndim)


def _mm2(x, wl, wr, bl):
    n = x.shape[0]
    h = wl.shape[1]
    din = x.shape[1]
    return pl.pallas_call(
        _mm2_body,
        grid=(n // _BR,),
        in_specs=[_row_spec((din,)), _full_spec(2), _full_spec(2), _full_spec(1)],
        out_specs=(_row_spec((h,)), _row_spec((h,))),
        out_shape=(jax.ShapeDtypeStruct((n, h), jnp.float32),
                   jax.ShapeDtypeStruct((n, h), jnp.float32)),
        compiler_params=_TC_PARAMS,
    )(x, wl, wr, bl)


def _epilogue_body(p_ref, cp_ref, z_ref, g_ref, b_ref, wln_ref, wrn_ref,
                   bln_ref, y_ref, zn_ref, inv_ref, h_ref):
    cnt = cp_ref[0] + cp_ref[1]
    inv = 1.0 / jnp.maximum(cnt, 1.0)
    inv_ref[...] = inv
    s = (p_ref[0] + p_ref[1]) * inv[:, 0:1] + z_ref[...]
    t = jnp.where(s >= 0, s, 0.1 * s)
    mu = jnp.mean(t, axis=-1, keepdims=True)
    var = jnp.mean(t * t, axis=-1, keepdims=True) - mu * mu
    hcur = ((t - mu) * lax.rsqrt(var + 1e-5) * g_ref[...][None, :]
            + b_ref[...][None, :])
    h_ref[...] = hcur
    y_ref[...] = jnp.dot(hcur, wln_ref[...], preferred_element_type=jnp.float32,
                         precision=lax.Precision.HIGHEST)
    zn_ref[...] = (jnp.dot(hcur, wrn_ref[...],
                           preferred_element_type=jnp.float32)
                   + bln_ref[...][None, :])


def _epilogue(p, cp, z, g, b, wln, wrn, bln):
    n = z.shape[0]
    h = z.shape[1]
    hn = wln.shape[1]
    return pl.pallas_call(
        _epilogue_body,
        grid=(n // _BR,),
        in_specs=[_part_spec((h,)), _part_spec((16,)), _row_spec((h,)),
                  _full_spec(1), _full_spec(1), _full_spec(2), _full_spec(2),
                  _full_spec(1)],
        out_specs=(_row_spec((hn,)), _row_spec((hn,)), _row_spec((16,)),
                   _row_spec((h,))),
        out_shape=(jax.ShapeDtypeStruct((n, hn), jnp.float32),
                   jax.ShapeDtypeStruct((n, hn), jnp.float32),
                   jax.ShapeDtypeStruct((n, 16), jnp.float32),
                   jax.ShapeDtypeStruct((n, h), jnp.float32)),
        compiler_params=_TC_PARAMS,
    )(p, cp, z, g, b, wln, wrn, bln)


def _epilogue2_body(p_ref, inv_ref, z_ref, g_ref, b_ref, wrn_ref, bln_ref,
                    h_ref, zn_ref):
    s = (p_ref[0] + p_ref[1]) * inv_ref[:, 0:1] + z_ref[...]
    t = jnp.where(s >= 0, s, 0.1 * s)
    mu = jnp.mean(t, axis=-1, keepdims=True)
    var = jnp.mean(t * t, axis=-1, keepdims=True) - mu * mu
    hcur = ((t - mu) * lax.rsqrt(var + 1e-5) * g_ref[...][None, :]
            + b_ref[...][None, :])
    h_ref[...] = hcur
    zn_ref[...] = (jnp.dot(hcur, wrn_ref[...],
                           preferred_element_type=jnp.float32)
                   + bln_ref[...][None, :])


def _epilogue2(p, inv, z, g, b, wrn, bln):
    n = z.shape[0]
    h = z.shape[1]
    hn = wrn.shape[1]
    return pl.pallas_call(
        _epilogue2_body,
        grid=(n // _BR,),
        in_specs=[_part_spec((h,)), _row_spec((16,)), _row_spec((h,)),
                  _full_spec(1), _full_spec(1), _full_spec(2), _full_spec(1)],
        out_specs=(_row_spec((h,)), _row_spec((hn,))),
        out_shape=(jax.ShapeDtypeStruct((n, h), jnp.float32),
                   jax.ShapeDtypeStruct((n, hn), jnp.float32)),
        compiler_params=_TC_PARAMS,
    )(p, inv, z, g, b, wrn, bln)


def _final_body(p_ref, inv_ref, wl_ref, z_ref, o_ref):
    agg = (p_ref[0] + p_ref[1]) * inv_ref[:, 0:1]
    o_ref[...] = (jnp.dot(agg, wl_ref[...], preferred_element_type=jnp.float32,
                          precision=lax.Precision.HIGHEST)
                  + z_ref[...])


def _final(p, inv, wl, z):
    n = z.shape[0]
    h = p.shape[2]
    dout = z.shape[1]
    return pl.pallas_call(
        _final_body,
        grid=(n // _BR,),
        in_specs=[_part_spec((h,)), _row_spec((16,)), _full_spec(2),
                  _row_spec((dout,))],
        out_specs=_row_spec((dout,)),
        out_shape=jax.ShapeDtypeStruct((n, dout), jnp.float32),
        compiler_params=_TC_PARAMS,
    )(p, inv, wl, z)


def kernel(x, edge_index, Wl1, bl1, Wr1, g1, b1, Wl2, bl2, Wr2, g2, b2,
           Wl3, bl3, Wr3):
    n = x.shape[0]
    e = edge_index.shape[1]
    h = Wl1.shape[1]

    n_chunks = -(-e // (_NW * _CH))
    n_chunks += n_chunks % 2
    e_pad = n_chunks * _CH * _NW
    n_acc = -(-(n + 1) // (_NS * 8)) * (_NS * 8)

    src = edge_index[0]
    dst = edge_index[1]
    pad = e_pad - e
    pad_i = jnp.arange(pad, dtype=jnp.int32)
    src_p = jnp.concatenate([src, pad_i % n]).reshape(_NW, n_chunks, _CH)
    dst_p = jnp.concatenate([dst, n + pad_i % (n_acc - n)]
                            ).reshape(_NW, n_chunks, _CH)
    z64 = jnp.zeros((n_acc, h), jnp.float32)
    z16 = jnp.zeros((n_acc, 16), jnp.float32)

    pass1 = _make_edge_pass(True, n_chunks, n_acc, n, h)
    pass2 = _make_edge_pass(False, n_chunks, n_acc, n, h)

    y1, z1 = _mm2(x, Wl1, Wr1, bl1)
    p1, c1 = pass1(y1, src_p, dst_p, z64, z16)
    y2, z2, inv, _h1 = _epilogue(p1, c1, z1, g1, b1, Wl2, Wr2, bl2)
    p2, = pass2(y2, src_p, dst_p, z64)
    h2, z3 = _epilogue2(p2, inv, z2, g2, b2, Wr3, bl3)
    p3, = pass2(h2, src_p, dst_p, z64)
    return _final(p3, inv, Wl3, z3)


# speedup vs baseline: 1.3334x; 1.3334x over previous
import functools

import jax
import jax.numpy as jnp
from jax import lax
from jax.experimental import pallas as pl
from jax.experimental.pallas import tpu as pltpu
from jax.experimental.pallas import tpu_sc as plsc

_NC = 2
_NS = 16
_NW = _NC * _NS
_CH = 128


@functools.lru_cache(maxsize=None)
def _make_edge_pass(with_count, n_chunks, n_acc, n, d):
    mesh = plsc.VectorSubcoreMesh(core_axis_name="c", subcore_axis_name="s")
    rps = n_acc // _NS

    out_type = [jax.ShapeDtypeStruct((_NC, n_acc, d), jnp.float32)]
    scratch = [
        pltpu.VMEM((n_chunks, _CH), jnp.int32),
        pltpu.VMEM((n_chunks, _CH), jnp.int32),
        pltpu.VMEM((_CH, d), jnp.float32),
        pltpu.VMEM((_CH, d), jnp.float32),
        pltpu.VMEM((_CH, d), jnp.float32),
        pltpu.VMEM((_CH, d), jnp.float32),
        pltpu.VMEM_SHARED((n_acc, d), jnp.float32),
        pltpu.SemaphoreType.DMA,
        pltpu.SemaphoreType.DMA,
        pltpu.SemaphoreType.DMA,
        pltpu.SemaphoreType.DMA,
    ]
    if with_count:
        out_type.append(jax.ShapeDtypeStruct((_NC, n_acc, 16), jnp.float32))
        scratch += [
            pltpu.VMEM((_CH, 16), jnp.float32),
            pltpu.VMEM_SHARED((n_acc, 16), jnp.float32),
        ]

    def striped(s, fn):
        fn(pl.ds(s * rps, rps))

    def body(y_ref, src_ref, dst_ref, z64_ref, *rest):
        if with_count:
            (z16_ref, part_ref, cnt_ref, src_v, dst_v, b0, b1, b2, b3,
             acc_sh, g0, g1, g2, g3, ones_v, cnt_sh) = rest
        else:
            (part_ref, src_v, dst_v, b0, b1, b2, b3, acc_sh,
             g0, g1, g2, g3) = rest
        c = lax.axis_index("c")
        s = lax.axis_index("s")
        wid = s * _NC + c

        striped(s, lambda sl: pltpu.sync_copy(z64_ref.at[sl], acc_sh.at[sl]))
        if with_count:
            striped(s, lambda sl: pltpu.sync_copy(z16_ref.at[sl],
                                                  cnt_sh.at[sl]))
            vec = jnp.where(lax.iota(jnp.int32, 16) == 0,
                            jnp.float32(1.0), jnp.float32(0.0))

            def init_ones(i, _):
                ones_v[i, :] = vec
                return 0
            lax.fori_loop(0, _CH, init_ones, 0)

        pltpu.sync_copy(src_ref.at[wid], src_v)
        pltpu.sync_copy(dst_ref.at[wid], dst_v)
        plsc.subcore_barrier()

        bufs = ((b0, g0), (b1, g1), (b2, g2), (b3, g3))
        for q, (buf, gsem) in enumerate(bufs):
            pltpu.async_copy(y_ref.at[src_v.at[q]], buf, gsem)

        def quad(j, _):
            a = 4 * j
            for q, (buf, gsem) in enumerate(bufs):
                k = a + q
                pltpu.make_async_copy(y_ref.at[src_v.at[k]], buf, gsem).wait()
                pltpu.sync_copy(buf, acc_sh.at[dst_v.at[k]], add=True)
                if with_count:
                    pltpu.sync_copy(ones_v, cnt_sh.at[dst_v.at[k]], add=True)

                @pl.when(k + 4 < n_chunks)
                def _():
                    pltpu.async_copy(y_ref.at[src_v.at[k + 4]], buf, gsem)
            return 0

        lax.fori_loop(0, n_chunks // 4, quad, 0)
        plsc.subcore_barrier()

        striped(s, lambda sl: pltpu.sync_copy(acc_sh.at[sl],
                                              part_ref.at[c, sl]))
        if with_count:
            striped(s, lambda sl: pltpu.sync_copy(cnt_sh.at[sl],
                                                  cnt_ref.at[c, sl]))

    return pl.kernel(body, out_type=tuple(out_type), mesh=mesh,
                     scratch_types=tuple(scratch),
                     compiler_params=pltpu.CompilerParams(
                         use_tc_tiling_on_sc=False))


def _mm2_body(x_ref, wl_ref, wr_ref, bl_ref, y_ref, z_ref):
    x = x_ref[...]
    y_ref[...] = jnp.dot(x, wl_ref[...], preferred_element_type=jnp.float32,
                         precision=lax.Precision.HIGHEST)
    z_ref[...] = (jnp.dot(x, wr_ref[...], preferred_element_type=jnp.float32)
                  + bl_ref[...][None, :])


_TC_PARAMS = pltpu.CompilerParams(vmem_limit_bytes=100 * 1024 * 1024)
_BR = 2000


def _row_spec(shape_tail):
    return pl.BlockSpec((_BR,) + shape_tail, lambda i: (i,) + (0,) * len(shape_tail))


def _part_spec(shape_tail):
    return pl.BlockSpec((2, _BR) + shape_tail,
                        lambda i: (0, i) + (0,) * len(shape_tail))


def _full_spec(ndim):
    return pl.BlockSpec(None, lambda i: (0,) * ndim)


def _mm2(x, wl, wr, bl):
    n = x.shape[0]
    h = wl.shape[1]
    din = x.shape[1]
    return pl.pallas_call(
        _mm2_body,
        grid=(n // _BR,),
        in_specs=[_row_spec((din,)), _full_spec(2), _full_spec(2), _full_spec(1)],
        out_specs=(_row_spec((h,)), _row_spec((h,))),
        out_shape=(jax.ShapeDtypeStruct((n, h), jnp.float32),
                   jax.ShapeDtypeStruct((n, h), jnp.float32)),
        compiler_params=_TC_PARAMS,
    )(x, wl, wr, bl)


def _epilogue_body(p_ref, cp_ref, z_ref, g_ref, b_ref, wln_ref, wrn_ref,
                   bln_ref, y_ref, zn_ref, inv_ref, h_ref):
    cnt = cp_ref[0] + cp_ref[1]
    inv = 1.0 / jnp.maximum(cnt, 1.0)
    inv_ref[...] = inv
    s = (p_ref[0] + p_ref[1]) * inv[:, 0:1] + z_ref[...]
    t = jnp.where(s >= 0, s, 0.1 * s)
    mu = jnp.mean(t, axis=-1, keepdims=True)
    var = jnp.mean(t * t, axis=-1, keepdims=True) - mu * mu
    hcur = ((t - mu) * lax.rsqrt(var + 1e-5) * g_ref[...][None, :]
            + b_ref[...][None, :])
    h_ref[...] = hcur
    y_ref[...] = jnp.dot(hcur, wln_ref[...], preferred_element_type=jnp.float32,
                         precision=lax.Precision.HIGHEST)
    zn_ref[...] = (jnp.dot(hcur, wrn_ref[...],
                           preferred_element_type=jnp.float32)
                   + bln_ref[...][None, :])


def _epilogue(p, cp, z, g, b, wln, wrn, bln):
    n = z.shape[0]
    h = z.shape[1]
    hn = wln.shape[1]
    return pl.pallas_call(
        _epilogue_body,
        grid=(n // _BR,),
        in_specs=[_part_spec((h,)), _part_spec((16,)), _row_spec((h,)),
                  _full_spec(1), _full_spec(1), _full_spec(2), _full_spec(2),
                  _full_spec(1)],
        out_specs=(_row_spec((hn,)), _row_spec((hn,)), _row_spec((16,)),
                   _row_spec((h,))),
        out_shape=(jax.ShapeDtypeStruct((n, hn), jnp.float32),
                   jax.ShapeDtypeStruct((n, hn), jnp.float32),
                   jax.ShapeDtypeStruct((n, 16), jnp.float32),
                   jax.ShapeDtypeStruct((n, h), jnp.float32)),
        compiler_params=_TC_PARAMS,
    )(p, cp, z, g, b, wln, wrn, bln)


def _epilogue2_body(p_ref, inv_ref, z_ref, g_ref, b_ref, wrn_ref, bln_ref,
                    h_ref, zn_ref):
    s = (p_ref[0] + p_ref[1]) * inv_ref[:, 0:1] + z_ref[...]
    t = jnp.where(s >= 0, s, 0.1 * s)
    mu = jnp.mean(t, axis=-1, keepdims=True)
    var = jnp.mean(t * t, axis=-1, keepdims=True) - mu * mu
    hcur = ((t - mu) * lax.rsqrt(var + 1e-5) * g_ref[...][None, :]
            + b_ref[...][None, :])
    h_ref[...] = hcur
    zn_ref[...] = (jnp.dot(hcur, wrn_ref[...],
                           preferred_element_type=jnp.float32)
                   + bln_ref[...][None, :])


def _epilogue2(p, inv, z, g, b, wrn, bln):
    n = z.shape[0]
    h = z.shape[1]
    hn = wrn.shape[1]
    return pl.pallas_call(
        _epilogue2_body,
        grid=(n // _BR,),
        in_specs=[_part_spec((h,)), _row_spec((16,)), _row_spec((h,)),
                  _full_spec(1), _full_spec(1), _full_spec(2), _full_spec(1)],
        out_specs=(_row_spec((h,)), _row_spec((hn,))),
        out_shape=(jax.ShapeDtypeStruct((n, h), jnp.float32),
                   jax.ShapeDtypeStruct((n, hn), jnp.float32)),
        compiler_params=_TC_PARAMS,
    )(p, inv, z, g, b, wrn, bln)


def _final_body(p_ref, inv_ref, wl_ref, z_ref, o_ref):
    agg = (p_ref[0] + p_ref[1]) * inv_ref[:, 0:1]
    o_ref[...] = (jnp.dot(agg, wl_ref[...], preferred_element_type=jnp.float32,
                          precision=lax.Precision.HIGHEST)
                  + z_ref[...])


def _final(p, inv, wl, z):
    n = z.shape[0]
    h = p.shape[2]
    dout = z.shape[1]
    return pl.pallas_call(
        _final_body,
        grid=(n // _BR,),
        in_specs=[_part_spec((h,)), _row_spec((16,)), _full_spec(2),
                  _row_spec((dout,))],
        out_specs=_row_spec((dout,)),
        out_shape=jax.ShapeDtypeStruct((n, dout), jnp.float32),
        compiler_params=_TC_PARAMS,
    )(p, inv, wl, z)


def kernel(x, edge_index, Wl1, bl1, Wr1, g1, b1, Wl2, bl2, Wr2, g2, b2,
           Wl3, bl3, Wr3):
    n = x.shape[0]
    e = edge_index.shape[1]
    h = Wl1.shape[1]

    n_chunks = -(-e // (_NW * _CH))
    n_chunks = -(-n_chunks // 4) * 4
    e_pad = n_chunks * _CH * _NW
    n_acc = -(-(n + 1) // (_NS * 8)) * (_NS * 8)

    src = edge_index[0]
    dst = edge_index[1]
    pad = e_pad - e
    pad_i = jnp.arange(pad, dtype=jnp.int32)
    src_p = jnp.concatenate([src, pad_i % n]).reshape(_NW, n_chunks, _CH)
    dst_p = jnp.concatenate([dst, n + pad_i % (n_acc - n)]
                            ).reshape(_NW, n_chunks, _CH)
    z64 = jnp.zeros((n_acc, h), jnp.float32)
    z16 = jnp.zeros((n_acc, 16), jnp.float32)

    pass1 = _make_edge_pass(True, n_chunks, n_acc, n, h)
    pass2 = _make_edge_pass(False, n_chunks, n_acc, n, h)

    y1, z1 = _mm2(x, Wl1, Wr1, bl1)
    p1, c1 = pass1(y1, src_p, dst_p, z64, z16)
    y2, z2, inv, _h1 = _epilogue(p1, c1, z1, g1, b1, Wl2, Wr2, bl2)
    p2, = pass2(y2, src_p, dst_p, z64)
    h2, z3 = _epilogue2(p2, inv, z2, g2, b2, Wr3, bl3)
    p3, = pass2(h2, src_p, dst_p, z64)
    return _final(p3, inv, Wl3, z3)
